# Initial kernel scaffold; baseline (speedup 1.0000x reference)
import jax
import jax.numpy as jnp
from jax.experimental import pallas as pl


def kernel(input0, input1, table):
    raise NotImplementedError("write your pallas kernel here")


# speedup vs baseline: 3.3123x; 3.3123x over previous
import functools

import jax
import jax.numpy as jnp
from jax import lax
from jax.experimental import pallas as pl
from jax.experimental.pallas import tpu as pltpu
from jax.experimental.pallas import tpu_sc as plsc

NC = 2
NS = 16
NW = NC * NS
L = 16

B = 4096 * 100
D = 10
BPW = B // NW
CHO = 1280
NCHO = BPW // CHO
GRP = CHO * D // (L * D)


def _ew_body(x_ref, o_ref):
    o_ref[...] = x_ref[...] * 0.5 + 2.0


@jax.jit
def _elementwise(input0):
    return pl.pallas_call(
        _ew_body,
        out_shape=jax.ShapeDtypeStruct(input0.shape, input0.dtype),
    )(input0)


def _gather_body(idx_hbm, table_hbm, out_hbm, idx_v, tbl_v, out_buf, sems):
    wid = lax.axis_index("s") * NC + lax.axis_index("c")
    pltpu.sync_copy(idx_hbm.at[wid], idx_v)
    pltpu.sync_copy(table_hbm, tbl_v)

    lane = lax.iota(jnp.int32, L)
    qoffs = [(ph * L + lane) // D for ph in range(D)]
    doffs = [(ph * L + lane) % D for ph in range(D)]

    copies = [None, None]
    for c in range(NCHO):
        slot = c % 2
        if copies[slot] is not None:
            copies[slot].wait()

        def grp(g, carry):
            base = c * CHO + g * L
            for ph in range(D):
                q = base + qoffs[ph]
                qv = plsc.load_gather(idx_v, [q])
                addr = qv * D + doffs[ph]
                t = plsc.load_gather(tbl_v, [addr])
                out_buf[slot, pl.ds(g * L * D + ph * L, L)] = t
            return carry

        lax.fori_loop(0, GRP, grp, 0)
        copies[slot] = pltpu.async_copy(
            out_buf.at[slot], out_hbm.at[wid, c], sems.at[slot]
        )
    for cp in copies:
        cp.wait()


@jax.jit
def _gather(idx, table_flat):
    mesh = plsc.VectorSubcoreMesh(core_axis_name="c", subcore_axis_name="s")
    f = functools.partial(
        pl.kernel,
        out_type=jax.ShapeDtypeStruct((NW, NCHO, CHO * D), jnp.float32),
        mesh=mesh,
        compiler_params=pltpu.CompilerParams(needs_layout_passes=False),
        scratch_types=[
            pltpu.VMEM((BPW,), jnp.int32),
            pltpu.VMEM((1024,), jnp.float32),
            pltpu.VMEM((2, CHO * D), jnp.float32),
            pltpu.SemaphoreType.DMA((2,)),
        ],
    )(_gather_body)
    return f(idx, table_flat)


def kernel(input0, input1, table):
    output0 = _elementwise(input0)
    idx = input1.astype(jnp.int32).reshape(NW, BPW)
    table_flat = jnp.pad(table.reshape(-1), (0, 24))
    output1 = _gather(idx, table_flat).reshape(4096, 100, D)
    return (output0, output1)


# speedup vs baseline: 10.3563x; 3.1267x over previous
import functools

import jax
import jax.numpy as jnp
from jax import lax
from jax.experimental import pallas as pl
from jax.experimental.pallas import tpu as pltpu
from jax.experimental.pallas import tpu_sc as plsc

NC = 2
NS = 16
NW = NC * NS
L = 16

N = 4096
J = 100
D = 10
R = J * D
IPW = N // NW
BPW = IPW * J
RC = 200
NRC = R // RC
ILV = IPW // L


def _ew_body(x_ref, o_ref):
    o_ref[...] = x_ref[...] * 0.5 + 2.0


@jax.jit
def _elementwise(input0):
    return pl.pallas_call(
        _ew_body,
        out_shape=jax.ShapeDtypeStruct(input0.shape, input0.dtype),
    )(input0)


def _gather_body(idx_hbm, table_hbm, out_hbm, idx_v, tbl_v, out_buf, sems):
    wid = lax.axis_index("s") * NC + lax.axis_index("c")
    i0 = wid * IPW
    pltpu.sync_copy(idx_hbm.at[pl.ds(wid * BPW, BPW)], idx_v)
    pltpu.sync_copy(table_hbm, tbl_v)

    lane100 = lax.iota(jnp.int32, L) * J
    ibase = [lane100 + (il * L * J) for il in range(ILV)]

    copies = [None, None]
    for c in range(NRC):
        slot = c % 2
        if copies[slot] is not None:
            copies[slot].wait()

        def row(rl, carry):
            r = c * RC + rl
            k = r // J
            j = r % J
            for il in range(ILV):
                a = ibase[il] + j
                qv = plsc.load_gather(idx_v, [a])
                t = plsc.load_gather(tbl_v, [qv * D + k])
                out_buf[slot, rl, pl.ds(il * L, L)] = t
            return carry

        lax.fori_loop(0, RC, row, 0)
        copies[slot] = pltpu.async_copy(
            out_buf.at[slot],
            out_hbm.at[pl.ds(c * RC, RC), pl.ds(i0, IPW)],
            sems.at[slot],
        )
    for cp in copies:
        cp.wait()


@jax.jit
def _gather(idx_flat, table_flat):
    mesh = plsc.VectorSubcoreMesh(core_axis_name="c", subcore_axis_name="s")
    f = functools.partial(
        pl.kernel,
        out_type=jax.ShapeDtypeStruct((R, N), jnp.float32),
        mesh=mesh,
        compiler_params=pltpu.CompilerParams(needs_layout_passes=False),
        scratch_types=[
            pltpu.VMEM((BPW,), jnp.int32),
            pltpu.VMEM((1024,), jnp.float32),
            pltpu.VMEM((2, RC, IPW), jnp.float32),
            pltpu.SemaphoreType.DMA((2,)),
        ],
    )(_gather_body)
    return f(idx_flat, table_flat)


def kernel(input0, input1, table):
    output0 = _elementwise(input0)
    idx_flat = input1.astype(jnp.int32).reshape(-1)
    table_flat = jnp.pad(table.reshape(-1), (0, 24))
    out_t = _gather(idx_flat, table_flat)
    output1 = jnp.transpose(out_t.reshape(D, J, N), (2, 1, 0))
    return (output0, output1)


# speedup vs baseline: 26.5124x; 2.5600x over previous
import functools

import jax
import jax.numpy as jnp
from jax import lax
from jax.experimental import pallas as pl
from jax.experimental.pallas import tpu as pltpu
from jax.experimental.pallas import tpu_sc as plsc

NC = 2
NS = 16
NW = NC * NS
L = 16

N = 4096
J = 100
D = 10
R = J * D
IPW = N // NW
BPW = IPW * J
RC = 200
NRC = R // RC
ILV = IPW // L


def _ew_body(x_ref, o_ref):
    o_ref[...] = x_ref[...] * 0.5 + 2.0


@jax.jit
def _elementwise(input0):
    return pl.pallas_call(
        _ew_body,
        out_shape=jax.ShapeDtypeStruct(input0.shape, input0.dtype),
    )(input0)


def _gather_body(idx_hbm, table_hbm, out_hbm, idx_v, tbl_v, out_buf, sems):
    wid = lax.axis_index("s") * NC + lax.axis_index("c")
    i0 = wid * IPW
    pltpu.sync_copy(idx_hbm.at[pl.ds(wid * BPW, BPW)], idx_v)
    pltpu.sync_copy(table_hbm, tbl_v)

    lane100 = lax.iota(jnp.int32, L) * J
    ibase = [lane100 + (il * L * J) for il in range(ILV)]

    copies = [None, None]
    for c in range(NRC):
        slot = c % 2
        if copies[slot] is not None:
            copies[slot].wait()

        @plsc.parallel_loop(0, RC, unroll=2)
        def row(rl):
            r = c * RC + rl
            k = r // J
            j = r % J
            for il in range(ILV):
                a = ibase[il] + j
                qv = plsc.load_gather(idx_v, [a])
                t = plsc.load_gather(tbl_v, [qv * D + k])
                out_buf[slot, rl, pl.ds(il * L, L)] = t
        copies[slot] = pltpu.async_copy(
            out_buf.at[slot],
            out_hbm.at[pl.ds(c * RC, RC), pl.ds(i0, IPW)],
            sems.at[slot],
        )
    for cp in copies:
        cp.wait()


@jax.jit
def _gather(idx_flat, table_flat):
    mesh = plsc.VectorSubcoreMesh(core_axis_name="c", subcore_axis_name="s")
    f = functools.partial(
        pl.kernel,
        out_type=jax.ShapeDtypeStruct((R, N), jnp.float32),
        mesh=mesh,
        compiler_params=pltpu.CompilerParams(needs_layout_passes=False),
        scratch_types=[
            pltpu.VMEM((BPW,), jnp.int32),
            pltpu.VMEM((1024,), jnp.float32),
            pltpu.VMEM((2, RC, IPW), jnp.float32),
            pltpu.SemaphoreType.DMA((2,)),
        ],
    )(_gather_body)
    return f(idx_flat, table_flat)


def kernel(input0, input1, table):
    output0 = _elementwise(input0)
    idx_flat = input1.astype(jnp.int32).reshape(-1)
    table_flat = jnp.pad(table.reshape(-1), (0, 24))
    out_t = _gather(idx_flat, table_flat)
    output1 = jnp.transpose(out_t.reshape(D, J, N), (2, 1, 0))
    return (output0, output1)


# speedup vs baseline: 28.2692x; 1.0663x over previous
import functools

import jax
import jax.numpy as jnp
from jax import lax
from jax.experimental import pallas as pl
from jax.experimental.pallas import tpu as pltpu
from jax.experimental.pallas import tpu_sc as plsc

NC = 2
NS = 16
NW = NC * NS
L = 16

N = 4096
J = 100
D = 10
R = J * D
IPW = N // NW
RC = 200
NRC = R // RC
ILV = IPW // L


def _ew_body(x_ref, o_ref):
    o_ref[...] = x_ref[...] * 0.5 + 2.0


@jax.jit
def _elementwise(input0):
    return pl.pallas_call(
        _ew_body,
        out_shape=jax.ShapeDtypeStruct(input0.shape, input0.dtype),
    )(input0)


def _gather_body(idx_hbm, table_hbm, out_hbm, idx_v, tbl_v, out_buf, sems):
    wid = lax.axis_index("s") * NC + lax.axis_index("c")
    i0 = wid * IPW
    pltpu.sync_copy(idx_hbm.at[:, pl.ds(i0, IPW)], idx_v)
    pltpu.sync_copy(table_hbm, tbl_v)

    copies = [None, None]
    for c in range(NRC):
        slot = c % 2
        if copies[slot] is not None:
            copies[slot].wait()

        @plsc.parallel_loop(0, RC, unroll=2)
        def row(rl):
            r = c * RC + rl
            k = r // J
            j = r % J
            for il in range(ILV):
                qv = idx_v[j, pl.ds(il * L, L)]
                t = plsc.load_gather(tbl_v, [qv * D + k])
                out_buf[slot, rl, pl.ds(il * L, L)] = t

        copies[slot] = pltpu.async_copy(
            out_buf.at[slot],
            out_hbm.at[pl.ds(c * RC, RC), pl.ds(i0, IPW)],
            sems.at[slot],
        )
    for cp in copies:
        cp.wait()


@jax.jit
def _gather(idx_t, table_flat):
    mesh = plsc.VectorSubcoreMesh(core_axis_name="c", subcore_axis_name="s")
    f = functools.partial(
        pl.kernel,
        out_type=jax.ShapeDtypeStruct((R, N), jnp.float32),
        mesh=mesh,
        compiler_params=pltpu.CompilerParams(needs_layout_passes=False),
        scratch_types=[
            pltpu.VMEM((J, IPW), jnp.int32),
            pltpu.VMEM((1024,), jnp.float32),
            pltpu.VMEM((2, RC, IPW), jnp.float32),
            pltpu.SemaphoreType.DMA((2,)),
        ],
    )(_gather_body)
    return f(idx_t, table_flat)


def kernel(input0, input1, table):
    output0 = _elementwise(input0)
    idx_t = input1.astype(jnp.int32).T
    table_flat = jnp.pad(table.reshape(-1), (0, 24))
    out_t = _gather(idx_t, table_flat)
    output1 = jnp.transpose(out_t.reshape(D, J, N), (2, 1, 0))
    return (output0, output1)


# speedup vs baseline: 30.5886x; 1.0820x over previous
import functools

import jax
import jax.numpy as jnp
from jax import lax
from jax.experimental import pallas as pl
from jax.experimental.pallas import tpu as pltpu
from jax.experimental.pallas import tpu_sc as plsc

NC = 2
NS = 16
NW = NC * NS
L = 16

N = 4096
J = 100
D = 10
R = J * D
IPW = N // NW
KCH = (4, 4, 2)
ILV = IPW // L
BUFR = max(KCH) * J


def _ew_body(x_ref, o_ref):
    o_ref[...] = x_ref[...] * 0.5 + 2.0


@jax.jit
def _elementwise(input0):
    return pl.pallas_call(
        _ew_body,
        out_shape=jax.ShapeDtypeStruct(input0.shape, input0.dtype),
    )(input0)


def _gather_body(idx_hbm, table_hbm, out_hbm, idx_v, tbl_v, out_buf, sems):
    wid = lax.axis_index("s") * NC + lax.axis_index("c")
    i0 = wid * IPW
    pltpu.sync_copy(idx_hbm.at[:, pl.ds(i0, IPW)], idx_v)
    pltpu.sync_copy(table_hbm, tbl_v)

    copies = [None, None]
    k0 = 0
    for c, kch in enumerate(KCH):
        slot = c % 2
        if copies[slot] is not None:
            copies[slot].wait()

        @plsc.parallel_loop(0, J, unroll=2)
        def row(j):
            for il in range(ILV):
                qv = idx_v[j, pl.ds(il * L, L)]
                ta = qv * D
                for kk in range(kch):
                    t = plsc.load_gather(tbl_v, [ta + (k0 + kk)])
                    out_buf[slot, kk * J + j, pl.ds(il * L, L)] = t

        copies[slot] = pltpu.async_copy(
            out_buf.at[slot, pl.ds(0, kch * J)],
            out_hbm.at[pl.ds(k0 * J, kch * J), pl.ds(i0, IPW)],
            sems.at[slot],
        )
        k0 += kch
    for cp in copies:
        if cp is not None:
            cp.wait()


@jax.jit
def _gather(idx_t, table_flat):
    mesh = plsc.VectorSubcoreMesh(core_axis_name="c", subcore_axis_name="s")
    f = functools.partial(
        pl.kernel,
        out_type=jax.ShapeDtypeStruct((R, N), jnp.float32),
        mesh=mesh,
        compiler_params=pltpu.CompilerParams(needs_layout_passes=False),
        scratch_types=[
            pltpu.VMEM((J, IPW), jnp.int32),
            pltpu.VMEM((1024,), jnp.float32),
            pltpu.VMEM((2, BUFR, IPW), jnp.float32),
            pltpu.SemaphoreType.DMA((2,)),
        ],
    )(_gather_body)
    return f(idx_t, table_flat)


def kernel(input0, input1, table):
    output0 = _elementwise(input0)
    idx_t = input1.astype(jnp.int32).T
    table_flat = jnp.pad(table.reshape(-1), (0, 24))
    out_t = _gather(idx_t, table_flat)
    output1 = jnp.transpose(out_t.reshape(D, J, N), (2, 1, 0))
    return (output0, output1)
